# Initial kernel scaffold; baseline (speedup 1.0000x reference)
import jax
import jax.numpy as jnp
from jax.experimental import pallas as pl


def kernel(x, edge_index, batch, W1, b1, W2, b2, W3, b3, Wl, bl):
    raise NotImplementedError("write your pallas kernel here")


# speedup vs baseline: 2.1280x; 2.1280x over previous
import functools
import jax
import jax.numpy as jnp
from jax import lax
from jax.experimental import pallas as pl

N = 100000
H = 128
B = 64
BN = 1000
NB = N // BN


def _scale_matmul_body(x_ref, deg_ref, w_ref, o_ref):
    dis = lax.rsqrt(deg_ref[...])
    h = jnp.dot(x_ref[...], w_ref[...], preferred_element_type=jnp.float32)
    o_ref[...] = h * dis


def _layer_body(agg_ref, g_ref, deg_ref, b_ref, w_ref, o_ref):
    dis = lax.rsqrt(deg_ref[...])
    h = jax.nn.relu((agg_ref[...] + g_ref[...]) * dis + b_ref[...])
    o_ref[...] = jnp.dot(h, w_ref[...], preferred_element_type=jnp.float32) * dis


def _pool_body(agg_ref, g_ref, deg_ref, b_ref, batch_ref, sums_ref, cnts_ref):
    i = pl.program_id(0)
    dis = lax.rsqrt(deg_ref[...])
    h = jax.nn.relu((agg_ref[...] + g_ref[...]) * dis + b_ref[...])
    seg = jnp.arange(B, dtype=jnp.int32)[None, :]
    oh = (batch_ref[...] == seg).astype(jnp.float32)
    part = lax.dot_general(oh, h, (((0,), (0,)), ((), ())),
                           preferred_element_type=jnp.float32)
    cpart = jnp.sum(oh, axis=0)[None, :]

    @pl.when(i == 0)
    def _():
        sums_ref[...] = jnp.zeros_like(sums_ref)
        cnts_ref[...] = jnp.zeros_like(cnts_ref)

    sums_ref[...] += part
    cnts_ref[...] += cpart


def _logits_body(sums_ref, cnts_ref, wl_ref, bl_ref, o_ref):
    cnt = jnp.maximum(cnts_ref[...], 1.0)
    pooled = sums_ref[...] / cnt[0, :, None]
    logits = jnp.dot(pooled, wl_ref[...], preferred_element_type=jnp.float32)
    logits = logits + bl_ref[...]
    m = jnp.max(logits, axis=1, keepdims=True)
    lse = m + jnp.log(jnp.sum(jnp.exp(logits - m), axis=1, keepdims=True))
    o_ref[...] = logits - lse


def _scale_matmul(x, deg, w):
    fin = x.shape[1]
    return pl.pallas_call(
        _scale_matmul_body,
        grid=(NB,),
        in_specs=[
            pl.BlockSpec((BN, fin), lambda i: (i, 0)),
            pl.BlockSpec((BN, 1), lambda i: (i, 0)),
            pl.BlockSpec((fin, H), lambda i: (0, 0)),
        ],
        out_specs=pl.BlockSpec((BN, H), lambda i: (i, 0)),
        out_shape=jax.ShapeDtypeStruct((N, H), jnp.float32),
    )(x, deg, w)


def _layer(agg, g, deg, b, w):
    return pl.pallas_call(
        _layer_body,
        grid=(NB,),
        in_specs=[
            pl.BlockSpec((BN, H), lambda i: (i, 0)),
            pl.BlockSpec((BN, H), lambda i: (i, 0)),
            pl.BlockSpec((BN, 1), lambda i: (i, 0)),
            pl.BlockSpec((1, H), lambda i: (0, 0)),
            pl.BlockSpec((H, H), lambda i: (0, 0)),
        ],
        out_specs=pl.BlockSpec((BN, H), lambda i: (i, 0)),
        out_shape=jax.ShapeDtypeStruct((N, H), jnp.float32),
    )(agg, g, deg, b, w)


def _pool(agg, g, deg, b, batch2):
    return pl.pallas_call(
        _pool_body,
        grid=(NB,),
        in_specs=[
            pl.BlockSpec((BN, H), lambda i: (i, 0)),
            pl.BlockSpec((BN, H), lambda i: (i, 0)),
            pl.BlockSpec((BN, 1), lambda i: (i, 0)),
            pl.BlockSpec((1, H), lambda i: (0, 0)),
            pl.BlockSpec((BN, 1), lambda i: (i, 0)),
        ],
        out_specs=[
            pl.BlockSpec((B, H), lambda i: (0, 0)),
            pl.BlockSpec((1, B), lambda i: (0, 0)),
        ],
        out_shape=[
            jax.ShapeDtypeStruct((B, H), jnp.float32),
            jax.ShapeDtypeStruct((1, B), jnp.float32),
        ],
    )(agg, g, deg, b, batch2)


def _logits(sums, cnts, wl, bl):
    return pl.pallas_call(
        _logits_body,
        in_specs=[
            pl.BlockSpec((B, H), lambda: (0, 0)),
            pl.BlockSpec((1, B), lambda: (0, 0)),
            pl.BlockSpec((H, 2), lambda: (0, 0)),
            pl.BlockSpec((1, 2), lambda: (0, 0)),
        ],
        out_specs=pl.BlockSpec((B, 2), lambda: (0, 0)),
        out_shape=jax.ShapeDtypeStruct((B, 2), jnp.float32),
    )(sums, cnts, wl, bl)


def kernel(x, edge_index, batch, W1, b1, W2, b2, W3, b3, Wl, bl):
    src = edge_index[0]
    dst = edge_index[1]
    deg = jnp.ones((N,), jnp.float32).at[dst].add(1.0)
    deg2 = deg[:, None]

    def aggregate(g):
        return jnp.zeros((N, H), jnp.float32).at[dst].add(g[src])

    g1 = _scale_matmul(x, deg2, W1)
    agg1 = aggregate(g1)
    g2 = _layer(agg1, g1, deg2, b1[None, :], W2)
    agg2 = aggregate(g2)
    g3 = _layer(agg2, g2, deg2, b2[None, :], W3)
    agg3 = aggregate(g3)
    sums, cnts = _pool(agg3, g3, deg2, b3[None, :], batch[:, None])
    return _logits(sums, cnts, Wl, bl[None, :])


# speedup vs baseline: 8.4005x; 3.9476x over previous
import functools
import jax
import jax.numpy as jnp
from jax import lax
from jax.experimental import pallas as pl
from jax.experimental.pallas import tpu as pltpu
from jax.experimental.pallas import tpu_sc as plsc

N = 100000
E = 1600000
H = 128
B = 64

BN = 1000
NB = N // BN

NC, NS = 2, 16
PADN = 104448
R = 13056
NPASS = 4
EC = 102400
EPAD = EC * NS
WIN = 2048
NWIN = EC // WIN
VPW = WIN // 16
CH = 64
STCAP = 144
WB = R // NS
DROWS = 12800
DTW = 16
DPW = 400
DNW = DPW // DTW
DSL = PADN // NS


def _scale_matmul_body(x_ref, deg_ref, w_ref, o_ref):
    dis = lax.rsqrt(deg_ref[...])
    h = jnp.dot(x_ref[...], w_ref[...], preferred_element_type=jnp.float32)
    o_ref[...] = h * dis


def _layer_body(agg_ref, g_ref, deg_ref, b_ref, w_ref, o_ref):
    dis = lax.rsqrt(deg_ref[...])
    h = jax.nn.relu((agg_ref[...] + g_ref[...]) * dis + b_ref[...])
    o_ref[...] = jnp.dot(h, w_ref[...], preferred_element_type=jnp.float32) * dis


def _pool_body(agg_ref, g_ref, deg_ref, b_ref, batch_ref, sums_ref, cnts_ref):
    i = pl.program_id(0)
    dis = lax.rsqrt(deg_ref[...])
    h = jax.nn.relu((agg_ref[...] + g_ref[...]) * dis + b_ref[...])
    seg = jnp.arange(B, dtype=jnp.int32)[None, :]
    oh = (batch_ref[...] == seg).astype(jnp.float32)
    part = lax.dot_general(oh, h, (((0,), (0,)), ((), ())),
                           preferred_element_type=jnp.float32)
    cpart = jnp.sum(oh, axis=0)[None, :]

    @pl.when(i == 0)
    def _():
        sums_ref[...] = jnp.zeros_like(sums_ref)
        cnts_ref[...] = jnp.zeros_like(cnts_ref)

    sums_ref[...] += part
    cnts_ref[...] += cpart


def _logits_body(sums_ref, cnts_ref, wl_ref, bl_ref, o_ref):
    cnt = jnp.maximum(cnts_ref[...], 1.0)
    pooled = sums_ref[...] / cnt[0, :, None]
    logits = jnp.dot(pooled, wl_ref[...], preferred_element_type=jnp.float32)
    logits = logits + bl_ref[...]
    m = jnp.max(logits, axis=1, keepdims=True)
    lse = m + jnp.log(jnp.sum(jnp.exp(logits - m), axis=1, keepdims=True))
    o_ref[...] = logits - lse


def _scale_matmul(x, deg, w):
    fin = x.shape[1]
    return pl.pallas_call(
        _scale_matmul_body,
        grid=(NB,),
        in_specs=[
            pl.BlockSpec((BN, fin), lambda i: (i, 0)),
            pl.BlockSpec((BN, 1), lambda i: (i, 0)),
            pl.BlockSpec((fin, H), lambda i: (0, 0)),
        ],
        out_specs=pl.BlockSpec((BN, H), lambda i: (i, 0)),
        out_shape=jax.ShapeDtypeStruct((N, H), jnp.float32),
    )(x, deg, w)


def _layer(agg, g, deg, b, w):
    return pl.pallas_call(
        _layer_body,
        grid=(NB,),
        in_specs=[
            pl.BlockSpec((BN, H), lambda i: (i, 0)),
            pl.BlockSpec((BN, H), lambda i: (i, 0)),
            pl.BlockSpec((BN, 1), lambda i: (i, 0)),
            pl.BlockSpec((1, H), lambda i: (0, 0)),
            pl.BlockSpec((H, H), lambda i: (0, 0)),
        ],
        out_specs=pl.BlockSpec((BN, H), lambda i: (i, 0)),
        out_shape=jax.ShapeDtypeStruct((N, H), jnp.float32),
    )(agg, g, deg, b, w)


def _pool(agg, g, deg, b, batch2):
    return pl.pallas_call(
        _pool_body,
        grid=(NB,),
        in_specs=[
            pl.BlockSpec((BN, H), lambda i: (i, 0)),
            pl.BlockSpec((BN, H), lambda i: (i, 0)),
            pl.BlockSpec((BN, 1), lambda i: (i, 0)),
            pl.BlockSpec((1, H), lambda i: (0, 0)),
            pl.BlockSpec((BN, 1), lambda i: (i, 0)),
        ],
        out_specs=[
            pl.BlockSpec((B, H), lambda i: (0, 0)),
            pl.BlockSpec((1, B), lambda i: (0, 0)),
        ],
        out_shape=[
            jax.ShapeDtypeStruct((B, H), jnp.float32),
            jax.ShapeDtypeStruct((1, B), jnp.float32),
        ],
    )(agg, g, deg, b, batch2)


def _logits(sums, cnts, wl, bl):
    return pl.pallas_call(
        _logits_body,
        in_specs=[
            pl.BlockSpec((B, H), lambda: (0, 0)),
            pl.BlockSpec((1, B), lambda: (0, 0)),
            pl.BlockSpec((H, 2), lambda: (0, 0)),
            pl.BlockSpec((1, 2), lambda: (0, 0)),
        ],
        out_specs=pl.BlockSpec((B, 2), lambda: (0, 0)),
        out_shape=jax.ShapeDtypeStruct((B, 2), jnp.float32),
    )(sums, cnts, wl, bl)


_MESH = plsc.VectorSubcoreMesh(core_axis_name="c", subcore_axis_name="s")


@functools.partial(
    pl.kernel,
    out_type=jax.ShapeDtypeStruct((2 * PADN,), jnp.float32),
    mesh=_MESH,
    compiler_params=pltpu.CompilerParams(needs_layout_passes=False),
    scratch_types=[
        pltpu.VMEM((DTW, 128), jnp.int32),
        pltpu.VMEM((DTW, 128), jnp.float32),
        pltpu.VMEM_SHARED((PADN,), jnp.float32),
    ],
)
def _sc_deg(dst2_hbm, ones_hbm, zeros1_hbm, deg_hbm, dwin, ones_v, acc):
    c = lax.axis_index("c")
    s = lax.axis_index("s")
    wid = s * NC + c

    pltpu.sync_copy(zeros1_hbm.at[pl.ds(s * DSL, DSL)], acc.at[pl.ds(s * DSL, DSL)])
    pltpu.sync_copy(ones_hbm, ones_v)
    plsc.subcore_barrier()

    def wbody(w, carry):
        row0 = wid * DPW + w * DTW
        pltpu.sync_copy(dst2_hbm.at[pl.ds(row0, DTW)], dwin)
        for j in range(DTW):
            pltpu.sync_copy(ones_v.at[j], acc.at[dwin.at[j]], add=True)
        return carry

    lax.fori_loop(0, DNW, wbody, jnp.int32(0))

    plsc.subcore_barrier()
    pltpu.sync_copy(acc.at[pl.ds(s * DSL, DSL)],
                    deg_hbm.at[pl.ds(c * PADN + s * DSL, DSL)])


@functools.partial(
    pl.kernel,
    out_type=jax.ShapeDtypeStruct((PADN, H), jnp.float32),
    mesh=_MESH,
    compiler_params=pltpu.CompilerParams(needs_layout_passes=False),
    scratch_types=[
        pltpu.VMEM((WIN,), jnp.int32),
        pltpu.VMEM((WIN,), jnp.int32),
        pltpu.VMEM((WIN,), jnp.int32),
        pltpu.VMEM((WIN,), jnp.int32),
        pltpu.VMEM((STCAP,), jnp.int32),
        pltpu.VMEM((STCAP,), jnp.int32),
        pltpu.VMEM((2, CH), jnp.int32),
        pltpu.VMEM((2, CH), jnp.int32),
        pltpu.VMEM((2, CH, H), jnp.float32),
        pltpu.VMEM_SHARED((R + 8, H), jnp.float32),
        pltpu.SemaphoreType.DMA,
        pltpu.SemaphoreType.DMA,
        pltpu.SemaphoreType.DMA,
        pltpu.SemaphoreType.DMA,
    ],
)
def _sc_aggregate(g_hbm, src_hbm, dst_hbm, zeros2_hbm, agg_hbm,
                  swin0, swin1, dwin0, dwin1, st_src, st_ldst, fsrc, fldst,
                  rows_v, acc, wsem0, wsem1, gsem, ssem):
    c = lax.axis_index("c")
    s = lax.axis_index("s")
    iota16 = lax.iota(jnp.int32, 16)
    c16 = lambda v: jnp.broadcast_to(v, (16,)).astype(jnp.int32)
    pad_src = iota16 * 64 + c16(s * 1024)
    wbufs = ((swin0, dwin0, wsem0), (swin1, dwin1, wsem1))

    def stream_win(w, slot):
        sw, dw, sem = wbufs[slot]
        ew = s * EC + w * WIN
        pltpu.async_copy(src_hbm.at[pl.ds(ew, WIN)], sw, sem)
        pltpu.async_copy(dst_hbm.at[pl.ds(ew, WIN)], dw, sem)

    def wait_win(slot):
        sw, dw, sem = wbufs[slot]
        pltpu.make_async_copy(src_hbm.at[pl.ds(0, WIN)], sw, sem).wait()
        pltpu.make_async_copy(dst_hbm.at[pl.ds(0, WIN)], dw, sem).wait()

    def wait_gather(slot):
        pltpu.make_async_copy(g_hbm.at[fsrc.at[slot]], rows_v.at[slot], gsem).wait()

    def start_scatter(slot):
        pltpu.async_copy(rows_v.at[slot], acc.at[fldst.at[slot]], ssem, add=True)

    def wait_scatter(slot):
        pltpu.make_async_copy(rows_v.at[slot], acc.at[fldst.at[slot]], ssem).wait()

    def fire(nf):
        r = nf & 1

        @pl.when(nf >= 2)
        def _():
            wait_scatter(r)

        @pl.when(nf >= 1)
        def _():
            wait_gather(1 - r)
            start_scatter(1 - r)

        for j in range(CH // 16):
            sl = pl.ds(j * 16, 16)
            fsrc[r, sl] = st_src[sl]
            fldst[r, sl] = st_ldst[sl]
        pltpu.async_copy(g_hbm.at[fsrc.at[r]], rows_v.at[r], gsem)
        st_src[pl.ds(0, 16)] = st_src[pl.ds(CH, 16)]
        st_ldst[pl.ds(0, 16)] = st_ldst[pl.ds(CH, 16)]

    def make_gbody(rbase, dw, sw):
        def gbody(gi, carry):
            cur_v, nf = carry
            for u in range(4):
                v = gi * 4 + u
                sl = pl.ds(v * 16, 16)
                d = dw[sl]
                sv = sw[sl]
                ld = d - c16(rbase)
                m = (ld >= jnp.zeros((16,), jnp.int32)) & (ld < jnp.full((16,), R, jnp.int32))
                mi = m.astype(jnp.int32)
                prefix = jnp.cumsum(mi)
                pos = cur_v + prefix - jnp.ones((16,), jnp.int32)
                plsc.store_scatter(st_ldst, [pos], ld, mask=m)
                plsc.store_scatter(st_src, [pos], sv, mask=m)
                cur_v = cur_v + plsc.all_reduce_population_count(m)
            cur_s = jnp.max(cur_v)
            full = cur_s >= CH
            pl.when(full)(lambda: fire(nf))
            cur_v = jnp.where(full, cur_v - jnp.full((16,), CH, jnp.int32), cur_v)
            nf = jnp.where(full, nf + 1, nf)
            return (cur_v, nf)
        return gbody

    for p in range(NPASS):
        rbase = (c * NPASS + p) * R

        pltpu.sync_copy(zeros2_hbm, acc.at[pl.ds(s * WB, WB)])
        plsc.subcore_barrier()
        stream_win(0, 0)

        def wbody(k, carry):
            w = k * 2
            stream_win(w + 1, 1)
            wait_win(0)
            carry = lax.fori_loop(0, VPW // 4, make_gbody(rbase, dwin0, swin0), carry)

            @pl.when(k + 1 < NWIN // 2)
            def _():
                stream_win(w + 2, 0)

            wait_win(1)
            return lax.fori_loop(0, VPW // 4, make_gbody(rbase, dwin1, swin1), carry)

        cur_v, nf = lax.fori_loop(0, NWIN // 2, wbody,
                                  (jnp.zeros((16,), jnp.int32), jnp.int32(0)))
        cur = jnp.max(cur_v)

        @pl.when(cur > 0)
        def _():
            for j in range(CH // 16):
                sl = pl.ds(j * 16, 16)
                lane = iota16 + jnp.full((16,), j * 16, jnp.int32)
                keep = lane < c16(cur)
                st_ldst[sl] = jnp.where(keep, st_ldst[sl], jnp.full((16,), R, jnp.int32))
                st_src[sl] = jnp.where(keep, st_src[sl], pad_src)
            fire(nf)

        nf = jnp.where(cur > 0, nf + 1, nf)

        @pl.when(nf >= 1)
        def _():
            rlast = (nf - 1) & 1
            wait_gather(rlast)
            start_scatter(rlast)

            @pl.when(nf >= 2)
            def _():
                wait_scatter(1 - rlast)

            wait_scatter(rlast)

        plsc.subcore_barrier()
        pltpu.sync_copy(acc.at[pl.ds(s * WB, WB)],
                        agg_hbm.at[pl.ds(rbase + s * WB, WB)])
        plsc.subcore_barrier()


def kernel(x, edge_index, batch, W1, b1, W2, b2, W3, b3, Wl, bl):
    src = edge_index[0]
    dst = edge_index[1]
    srcp = jnp.concatenate([src, jnp.zeros((EPAD - E,), jnp.int32)])
    dstp = jnp.concatenate([dst, jnp.full((EPAD - E,), 1 << 29, jnp.int32)])
    dstp = jnp.concatenate([dst, jnp.full((DROWS * 128 - E,), N, jnp.int32)])
    dst2 = dstp.reshape(DROWS, 128)
    ones_h = jnp.ones((DTW, 128), jnp.float32)
    zeros1 = jnp.zeros((PADN,), jnp.float32)
    zeros2 = jnp.zeros((WB, H), jnp.float32)

    degp = _sc_deg(dst2, ones_h, zeros1)
    deg2 = (degp[:N] + degp[PADN:PADN + N])[:, None] + 1.0

    g1 = _scale_matmul(x, deg2, W1)
    agg1 = _sc_aggregate(g1, srcp, dstp, zeros2)
    g2 = _layer(agg1, g1, deg2, b1[None, :], W2)
    agg2 = _sc_aggregate(g2, srcp, dstp, zeros2)
    g3 = _layer(agg2, g2, deg2, b2[None, :], W3)
    agg3 = _sc_aggregate(g3, srcp, dstp, zeros2)
    sums, cnts = _pool(agg3, g3, deg2, b3[None, :], batch[:, None])
    return _logits(sums, cnts, Wl, bl[None, :])
